# baseline (speedup 1.0000x reference)
import functools

import jax
import jax.numpy as jnp
from jax import lax
from jax.experimental import pallas as pl
from jax.experimental.pallas import tpu as pltpu
from jax.experimental.pallas import tpu_sc as plsc

B, N, F_IN, H = 64, 512, 512, 768
ROWS = B * N

_NC, _NS = 2, 16
_NW = _NC * _NS
_ROWS_PER_W = ROWS // _NW
_CHUNK = 64
_NCHUNK = _ROWS_PER_W // _CHUNK


def _sc_gather_body(in_table, out_table, din_hbm, dout_hbm,
                    gin_hbm, gout_hbm,
                    idx_a, idx_b, buf_a, buf_b, sem_a, sem_b):
    wid = lax.axis_index("s") * _NC + lax.axis_index("c")
    base = wid * _ROWS_PER_W

    def chunk(c, _):
        off = base + c * _CHUNK
        pltpu.sync_copy(din_hbm.at[pl.ds(off, _CHUNK)], idx_a)
        pltpu.sync_copy(dout_hbm.at[pl.ds(off, _CHUNK)], idx_b)
        cp_a = pltpu.async_copy(in_table.at[idx_a], buf_a, sem_a)
        cp_b = pltpu.async_copy(out_table.at[idx_b], buf_b, sem_b)
        cp_a.wait()
        cp_b.wait()
        wr_a = pltpu.async_copy(buf_a, gin_hbm.at[pl.ds(off, _CHUNK)], sem_a)
        wr_b = pltpu.async_copy(buf_b, gout_hbm.at[pl.ds(off, _CHUNK)], sem_b)
        wr_a.wait()
        wr_b.wait()
        return ()

    lax.fori_loop(0, _NCHUNK, chunk, (), unroll=False)


_sc_gather = pl.kernel(
    _sc_gather_body,
    out_type=(
        jax.ShapeDtypeStruct((ROWS, H), jnp.float32),
        jax.ShapeDtypeStruct((ROWS, H), jnp.float32),
    ),
    mesh=plsc.VectorSubcoreMesh(core_axis_name="c", subcore_axis_name="s"),
    scratch_types=[
        pltpu.VMEM((_CHUNK,), jnp.int32),
        pltpu.VMEM((_CHUNK,), jnp.int32),
        pltpu.VMEM((_CHUNK, H), jnp.float32),
        pltpu.VMEM((_CHUNK, H), jnp.float32),
        pltpu.SemaphoreType.DMA,
        pltpu.SemaphoreType.DMA,
    ],
)


def _mm_body(x_ref, w_ref, b_ref, gin_ref, gout_ref, o_ref):
    acc = jnp.dot(x_ref[...].astype(jnp.bfloat16),
                  w_ref[...].astype(jnp.bfloat16),
                  preferred_element_type=jnp.float32)
    o_ref[...] = acc + b_ref[...] + gin_ref[...] + gout_ref[...]


_BM = 512


def _tc_matmul(x2, w, b, gin, gout):
    grid = (ROWS // _BM,)
    return pl.pallas_call(
        _mm_body,
        grid=grid,
        in_specs=[
            pl.BlockSpec((_BM, F_IN), lambda i: (i, 0)),
            pl.BlockSpec((F_IN, H), lambda i: (0, 0)),
            pl.BlockSpec((1, H), lambda i: (0, 0)),
            pl.BlockSpec((_BM, H), lambda i: (i, 0)),
            pl.BlockSpec((_BM, H), lambda i: (i, 0)),
        ],
        out_specs=pl.BlockSpec((_BM, H), lambda i: (i, 0)),
        out_shape=jax.ShapeDtypeStruct((ROWS, H), jnp.float32),
    )(x2, w, b, gin, gout)


def kernel(x, in_degree, out_degree, W_node, b_node, in_table, out_table):
    x2 = x.reshape(ROWS, F_IN)
    din = in_degree.reshape(ROWS).astype(jnp.int32)
    dout = out_degree.reshape(ROWS).astype(jnp.int32)
    gin, gout = _sc_gather(in_table, out_table, din, dout)
    out = _tc_matmul(x2, W_node, b_node.reshape(1, H), gin, gout)
    return out.reshape(B, N, H)


# speedup vs baseline: 1.0271x; 1.0271x over previous
import functools

import jax
import jax.numpy as jnp
from jax import lax
from jax.experimental import pallas as pl
from jax.experimental.pallas import tpu as pltpu
from jax.experimental.pallas import tpu_sc as plsc

B, N, F_IN, H = 64, 512, 512, 768
ROWS = B * N

_NC, _NS = 2, 16
_NW = _NC * _NS
_ROWS_PER_W = ROWS // _NW
_CHUNK = 32
_NCHUNK = _ROWS_PER_W // _CHUNK
_NBUF = 2


def _sc_gather_body(in_table, out_table, din_hbm, dout_hbm,
                    gin_hbm, gout_hbm,
                    idx_a, idx_b, bufs_a, bufs_b, gsems_a, gsems_b,
                    wsems_a, wsems_b):
    wid = lax.axis_index("s") * _NC + lax.axis_index("c")
    base = wid * _ROWS_PER_W

    pltpu.sync_copy(din_hbm.at[pl.ds(base, _ROWS_PER_W)], idx_a)
    pltpu.sync_copy(dout_hbm.at[pl.ds(base, _ROWS_PER_W)], idx_b)

    def start_gather(c, b):
        s = pl.ds(c * _CHUNK, _CHUNK)
        pltpu.async_copy(in_table.at[idx_a.at[s]], bufs_a.at[b], gsems_a[b])
        pltpu.async_copy(out_table.at[idx_b.at[s]], bufs_b.at[b], gsems_b[b])

    def wait_gather(b):
        pltpu.make_async_copy(in_table.at[idx_a.at[pl.ds(0, _CHUNK)]],
                              bufs_a.at[b], gsems_a[b]).wait()
        pltpu.make_async_copy(out_table.at[idx_b.at[pl.ds(0, _CHUNK)]],
                              bufs_b.at[b], gsems_b[b]).wait()

    def start_write(c, b):
        off = base + c * _CHUNK
        pltpu.async_copy(bufs_a.at[b], gin_hbm.at[pl.ds(off, _CHUNK)],
                         wsems_a[b])
        pltpu.async_copy(bufs_b.at[b], gout_hbm.at[pl.ds(off, _CHUNK)],
                         wsems_b[b])

    def wait_write(b):
        pltpu.make_async_copy(bufs_a.at[b], gin_hbm.at[pl.ds(0, _CHUNK)],
                              wsems_a[b]).wait()
        pltpu.make_async_copy(bufs_b.at[b], gout_hbm.at[pl.ds(0, _CHUNK)],
                              wsems_b[b]).wait()

    for b in range(_NBUF):
        start_gather(b, b)

    def pair(g, _):
        for b in range(_NBUF):
            c = _NBUF * g + b
            wait_gather(b)
            start_write(c, b)
        for b in range(_NBUF):
            c = _NBUF * g + b
            wait_write(b)

            @pl.when(c + _NBUF < _NCHUNK)
            def _():
                start_gather(c + _NBUF, b)
        return ()

    lax.fori_loop(0, _NCHUNK // _NBUF, pair, (), unroll=False)


_sc_gather = pl.kernel(
    _sc_gather_body,
    out_type=(
        jax.ShapeDtypeStruct((ROWS, H), jnp.float32),
        jax.ShapeDtypeStruct((ROWS, H), jnp.float32),
    ),
    mesh=plsc.VectorSubcoreMesh(core_axis_name="c", subcore_axis_name="s"),
    scratch_types=[
        pltpu.VMEM((_ROWS_PER_W,), jnp.int32),
        pltpu.VMEM((_ROWS_PER_W,), jnp.int32),
        pltpu.VMEM((_NBUF, _CHUNK, H), jnp.float32),
        pltpu.VMEM((_NBUF, _CHUNK, H), jnp.float32),
        [pltpu.SemaphoreType.DMA] * _NBUF,
        [pltpu.SemaphoreType.DMA] * _NBUF,
        [pltpu.SemaphoreType.DMA] * _NBUF,
        [pltpu.SemaphoreType.DMA] * _NBUF,
    ],
)


def _mm_body(x_ref, w_ref, b_ref, gin_ref, gout_ref, o_ref):
    acc = jnp.dot(x_ref[...].astype(jnp.bfloat16),
                  w_ref[...].astype(jnp.bfloat16),
                  preferred_element_type=jnp.float32)
    o_ref[...] = acc + b_ref[...] + gin_ref[...] + gout_ref[...]


_BM = 512


def _tc_matmul(x2, w, b, gin, gout):
    grid = (ROWS // _BM,)
    return pl.pallas_call(
        _mm_body,
        grid=grid,
        in_specs=[
            pl.BlockSpec((_BM, F_IN), lambda i: (i, 0)),
            pl.BlockSpec((F_IN, H), lambda i: (0, 0)),
            pl.BlockSpec((1, H), lambda i: (0, 0)),
            pl.BlockSpec((_BM, H), lambda i: (i, 0)),
            pl.BlockSpec((_BM, H), lambda i: (i, 0)),
        ],
        out_specs=pl.BlockSpec((_BM, H), lambda i: (i, 0)),
        out_shape=jax.ShapeDtypeStruct((ROWS, H), jnp.float32),
    )(x2, w, b, gin, gout)


def kernel(x, in_degree, out_degree, W_node, b_node, in_table, out_table):
    x2 = x.reshape(ROWS, F_IN)
    din = in_degree.reshape(ROWS).astype(jnp.int32)
    dout = out_degree.reshape(ROWS).astype(jnp.int32)
    gin, gout = _sc_gather(in_table, out_table, din, dout)
    out = _tc_matmul(x2, W_node, b_node.reshape(1, H), gin, gout)
    return out.reshape(B, N, H)


# speedup vs baseline: 1.4174x; 1.3801x over previous
import functools

import jax
import jax.numpy as jnp
from jax import lax
from jax.experimental import pallas as pl
from jax.experimental.pallas import tpu as pltpu
from jax.experimental.pallas import tpu_sc as plsc

B, N, F_IN, H = 64, 512, 512, 768
ROWS = B * N

_NC, _NS = 2, 16
_NW = _NC * _NS
_ROWS_PER_W = ROWS // _NW
_CHUNK = 64
_NCHUNK = _ROWS_PER_W // _CHUNK
_NBUF = 2


def _sc_gather_body(in_table, out_table, din_hbm, dout_hbm,
                    gin_hbm, gout_hbm,
                    idx_a, idx_b, bufs_a, bufs_b, gsems_a, gsems_b,
                    wsems_a, wsems_b):
    wid = lax.axis_index("s") * _NC + lax.axis_index("c")
    base = wid * _ROWS_PER_W

    pltpu.sync_copy(din_hbm.at[pl.ds(base, _ROWS_PER_W)], idx_a)
    pltpu.sync_copy(dout_hbm.at[pl.ds(base, _ROWS_PER_W)], idx_b)

    def start_gather(c, b):
        s = pl.ds(c * _CHUNK, _CHUNK)
        pltpu.async_copy(in_table.at[idx_a.at[s]], bufs_a.at[b], gsems_a[b])
        pltpu.async_copy(out_table.at[idx_b.at[s]], bufs_b.at[b], gsems_b[b])

    def wait_gather(b):
        pltpu.make_async_copy(in_table.at[idx_a.at[pl.ds(0, _CHUNK)]],
                              bufs_a.at[b], gsems_a[b]).wait()
        pltpu.make_async_copy(out_table.at[idx_b.at[pl.ds(0, _CHUNK)]],
                              bufs_b.at[b], gsems_b[b]).wait()

    def start_write(c, b):
        off = base + c * _CHUNK
        pltpu.async_copy(bufs_a.at[b], gin_hbm.at[pl.ds(off, _CHUNK)],
                         wsems_a[b])
        pltpu.async_copy(bufs_b.at[b], gout_hbm.at[pl.ds(off, _CHUNK)],
                         wsems_b[b])

    def wait_write(b):
        pltpu.make_async_copy(bufs_a.at[b], gin_hbm.at[pl.ds(0, _CHUNK)],
                              wsems_a[b]).wait()
        pltpu.make_async_copy(bufs_b.at[b], gout_hbm.at[pl.ds(0, _CHUNK)],
                              wsems_b[b]).wait()

    for b in range(_NBUF):
        start_gather(b, b)

    def pair(g, _):
        for b in range(_NBUF):
            c = _NBUF * g + b
            wait_gather(b)
            start_write(c, b)
        for b in range(_NBUF):
            c = _NBUF * g + b
            wait_write(b)

            @pl.when(c + _NBUF < _NCHUNK)
            def _():
                start_gather(c + _NBUF, b)
        return ()

    lax.fori_loop(0, _NCHUNK // _NBUF, pair, (), unroll=False)


_sc_gather = pl.kernel(
    _sc_gather_body,
    out_type=(
        jax.ShapeDtypeStruct((ROWS, H // 2), jnp.int32),
        jax.ShapeDtypeStruct((ROWS, H // 2), jnp.int32),
    ),
    mesh=plsc.VectorSubcoreMesh(core_axis_name="c", subcore_axis_name="s"),
    scratch_types=[
        pltpu.VMEM((_ROWS_PER_W,), jnp.int32),
        pltpu.VMEM((_ROWS_PER_W,), jnp.int32),
        pltpu.VMEM((_NBUF, _CHUNK, H // 2), jnp.int32),
        pltpu.VMEM((_NBUF, _CHUNK, H // 2), jnp.int32),
        [pltpu.SemaphoreType.DMA] * _NBUF,
        [pltpu.SemaphoreType.DMA] * _NBUF,
        [pltpu.SemaphoreType.DMA] * _NBUF,
        [pltpu.SemaphoreType.DMA] * _NBUF,
    ],
)


def _unpack_lo_hi(g):
    lo = lax.bitcast_convert_type(g << 16, jnp.float32)
    hi = lax.bitcast_convert_type(g & jnp.int32(-65536), jnp.float32)
    return lo, hi


def _mm_body(x_ref, w_ref, b_ref, gin_ref, gout_ref, o_ref):
    acc = jnp.dot(x_ref[...].astype(jnp.bfloat16),
                  w_ref[...].astype(jnp.bfloat16),
                  preferred_element_type=jnp.float32)
    acc = acc + b_ref[...]
    lo_i, hi_i = _unpack_lo_hi(gin_ref[...])
    lo_o, hi_o = _unpack_lo_hi(gout_ref[...])
    o_ref[:, : H // 2] = acc[:, : H // 2] + lo_i + lo_o
    o_ref[:, H // 2:] = acc[:, H // 2:] + hi_i + hi_o


_BM = 512


def _tc_matmul(x2, w, b, gin, gout):
    grid = (ROWS // _BM,)
    return pl.pallas_call(
        _mm_body,
        grid=grid,
        in_specs=[
            pl.BlockSpec((_BM, F_IN), lambda i: (i, 0)),
            pl.BlockSpec((F_IN, H), lambda i: (0, 0)),
            pl.BlockSpec((1, H), lambda i: (0, 0)),
            pl.BlockSpec((_BM, H // 2), lambda i: (i, 0)),
            pl.BlockSpec((_BM, H // 2), lambda i: (i, 0)),
        ],
        out_specs=pl.BlockSpec((_BM, H), lambda i: (i, 0)),
        out_shape=jax.ShapeDtypeStruct((ROWS, H), jnp.float32),
    )(x2, w, b, gin, gout)


def _pack_table(t):
    u = lax.bitcast_convert_type(t.astype(jnp.bfloat16), jnp.uint16)
    u = u.astype(jnp.uint32)
    packed = u[:, : H // 2] | (u[:, H // 2:] << 16)
    return lax.bitcast_convert_type(packed, jnp.int32)


def kernel(x, in_degree, out_degree, W_node, b_node, in_table, out_table):
    x2 = x.reshape(ROWS, F_IN)
    din = in_degree.reshape(ROWS).astype(jnp.int32)
    dout = out_degree.reshape(ROWS).astype(jnp.int32)
    gin, gout = _sc_gather(_pack_table(in_table), _pack_table(out_table),
                           din, dout)
    out = _tc_matmul(x2, W_node, b_node.reshape(1, H), gin, gout)
    return out.reshape(B, N, H)
